# baseline (speedup 1.0000x reference)
import functools

import jax
import jax.numpy as jnp
from jax.experimental import pallas as pl
from jax.experimental.pallas import tpu as pltpu

B = 2
S = 2048
H = 2
D_HEAD = 128
D_MODEL = 2048
E = 8
N_TOK = B * S
TM = 256


def _body(x_ref, sel_ref, p_ref, w_ref, b_ref, o_ref):
    x = x_ref[...]
    sa = sel_ref[:, 0:1]
    sb = sel_ref[:, 1:2]
    pa = p_ref[:, 0:1]
    pb = p_ref[:, 1:2]
    xa = x[:, :D_HEAD] * pa
    xb = x[:, D_HEAD:] * pb

    parts = []
    bw_cols = []
    for e in range(E):
        parts.append((jnp.where(sa == e, xa, 0.0)
                      + jnp.where(sb == e, xb, 0.0)).astype(jnp.bfloat16))
        bw_cols.append(jnp.where(sa == e, pa, 0.0) + jnp.where(sb == e, pb, 0.0))
    xd = jnp.concatenate(parts, axis=1)
    bw = jnp.concatenate(bw_cols, axis=1)
    acc = jax.lax.dot_general(
        bw, b_ref[...], (((1,), (0,)), ((), ())),
        preferred_element_type=jnp.float32)
    acc = acc + jax.lax.dot_general(
        xd, w_ref[...], (((1,), (0,)), ((), ())),
        preferred_element_type=jnp.float32)
    o_ref[...] = acc


@functools.partial(jax.jit, static_argnames=("interpret",))
def kernel(tensor, head_selection, head_probabilities, W, b, interpret=False):
    x2 = tensor.reshape(N_TOK, H * D_HEAD)
    sel = head_selection.reshape(N_TOK, H)
    p = head_probabilities.reshape(N_TOK, H)
    w2 = W.reshape(E * D_HEAD, D_MODEL).astype(jnp.bfloat16)

    grid = (N_TOK // TM,)
    out = pl.pallas_call(
        _body,
        grid=grid,
        in_specs=[
            pl.BlockSpec((TM, H * D_HEAD), lambda i: (i, 0)),
            pl.BlockSpec((TM, H), lambda i: (i, 0)),
            pl.BlockSpec((TM, H), lambda i: (i, 0)),
            pl.BlockSpec((E * D_HEAD, D_MODEL), lambda i: (0, 0)),
            pl.BlockSpec((E, D_MODEL), lambda i: (0, 0)),
        ],
        out_specs=pl.BlockSpec((TM, D_MODEL), lambda i: (i, 0)),
        out_shape=jax.ShapeDtypeStruct((N_TOK, D_MODEL), jnp.float32),
        compiler_params=pltpu.CompilerParams(
            dimension_semantics=("arbitrary",),
        ),
        interpret=interpret,
    )(x2, sel, p, w2, b)
    return out.reshape(B, S, D_MODEL)


# speedup vs baseline: 1.1219x; 1.1219x over previous
import functools

import jax
import jax.numpy as jnp
from jax.experimental import pallas as pl
from jax.experimental.pallas import tpu as pltpu

B = 2
S = 2048
H = 2
D_HEAD = 128
D_MODEL = 2048
E = 8
N_TOK = B * S
TM = 256


def _body(x_ref, sel_ref, p_ref, w_ref, b_ref, o_ref, wbf_ref):
    @pl.when(pl.program_id(0) == 0)
    def _cast_w():
        for j in range(E // 2):
            wbf_ref[j] = w_ref[j].astype(jnp.bfloat16)

    x = x_ref[...]
    sa = sel_ref[:, 0:1]
    sb = sel_ref[:, 1:2]
    pa = p_ref[:, 0:1]
    pb = p_ref[:, 1:2]
    xa = x[:, :D_HEAD] * pa
    xb = x[:, D_HEAD:] * pb

    bw_cols = []
    for e in range(E):
        bw_cols.append(jnp.where(sa == e, pa, 0.0) + jnp.where(sb == e, pb, 0.0))
    bw = jnp.concatenate(bw_cols, axis=1)
    acc = jax.lax.dot_general(
        bw, b_ref[...], (((1,), (0,)), ((), ())),
        preferred_element_type=jnp.float32)
    for j in range(E // 2):
        e0, e1 = 2 * j, 2 * j + 1
        xm0 = jnp.where(sa == e0, xa, 0.0) + jnp.where(sb == e0, xb, 0.0)
        xm1 = jnp.where(sa == e1, xa, 0.0) + jnp.where(sb == e1, xb, 0.0)
        xm = jnp.concatenate([xm0, xm1], axis=1).astype(jnp.bfloat16)
        acc = acc + jax.lax.dot_general(
            xm, wbf_ref[j],
            (((1,), (0,)), ((), ())),
            preferred_element_type=jnp.float32)
    o_ref[...] = acc


@functools.partial(jax.jit, static_argnames=("interpret",))
def kernel(tensor, head_selection, head_probabilities, W, b, interpret=False):
    x2 = tensor.reshape(N_TOK, H * D_HEAD)
    sel = head_selection.reshape(N_TOK, H)
    p = head_probabilities.reshape(N_TOK, H)
    w4 = W.reshape(E // 2, 2 * D_HEAD, D_MODEL)

    grid = (N_TOK // TM,)
    out = pl.pallas_call(
        _body,
        grid=grid,
        in_specs=[
            pl.BlockSpec((TM, H * D_HEAD), lambda i: (i, 0)),
            pl.BlockSpec((TM, H), lambda i: (i, 0)),
            pl.BlockSpec((TM, H), lambda i: (i, 0)),
            pl.BlockSpec((E // 2, 2 * D_HEAD, D_MODEL), lambda i: (0, 0, 0)),
            pl.BlockSpec((E, D_MODEL), lambda i: (0, 0)),
        ],
        out_specs=pl.BlockSpec((TM, D_MODEL), lambda i: (i, 0)),
        out_shape=jax.ShapeDtypeStruct((N_TOK, D_MODEL), jnp.float32),
        scratch_shapes=[pltpu.VMEM((E // 2, 2 * D_HEAD, D_MODEL), jnp.bfloat16)],
        compiler_params=pltpu.CompilerParams(
            dimension_semantics=("arbitrary",),
        ),
        interpret=interpret,
    )(x2, sel, p, w4, b)
    return out.reshape(B, S, D_MODEL)


# speedup vs baseline: 1.1488x; 1.0239x over previous
import functools

import jax
import jax.numpy as jnp
from jax.experimental import pallas as pl
from jax.experimental.pallas import tpu as pltpu

B = 2
S = 2048
H = 2
D_HEAD = 128
D_MODEL = 2048
E = 8
N_TOK = B * S
TM = 512


def _body(x_ref, sel_ref, p_ref, w_ref, b_ref, o_ref, wbf_ref):
    @pl.when(pl.program_id(0) == 0)
    def _cast_w():
        for j in range(E // 2):
            wbf_ref[j] = w_ref[j].astype(jnp.bfloat16)

    x = x_ref[...]
    sa = sel_ref[:, 0:1]
    sb = sel_ref[:, 1:2]
    pa = p_ref[:, 0:1]
    pb = p_ref[:, 1:2]
    xa = x[:, :D_HEAD] * pa
    xb = x[:, D_HEAD:] * pb

    bw_cols = []
    for e in range(E):
        bw_cols.append(jnp.where(sa == e, pa, 0.0) + jnp.where(sb == e, pb, 0.0))
    bw = jnp.concatenate(bw_cols, axis=1)
    acc = jax.lax.dot_general(
        bw, b_ref[...], (((1,), (0,)), ((), ())),
        preferred_element_type=jnp.float32)
    for j in range(E // 2):
        e0, e1 = 2 * j, 2 * j + 1
        xm0 = jnp.where(sa == e0, xa, 0.0) + jnp.where(sb == e0, xb, 0.0)
        xm1 = jnp.where(sa == e1, xa, 0.0) + jnp.where(sb == e1, xb, 0.0)
        xm = jnp.concatenate([xm0, xm1], axis=1).astype(jnp.bfloat16)
        acc = acc + jax.lax.dot_general(
            xm, wbf_ref[j],
            (((1,), (0,)), ((), ())),
            preferred_element_type=jnp.float32)
    o_ref[...] = acc


@functools.partial(jax.jit, static_argnames=("interpret",))
def kernel(tensor, head_selection, head_probabilities, W, b, interpret=False):
    x2 = tensor.reshape(N_TOK, H * D_HEAD)
    sel = head_selection.reshape(N_TOK, H)
    p = head_probabilities.reshape(N_TOK, H)
    w4 = W.reshape(E // 2, 2 * D_HEAD, D_MODEL)

    grid = (N_TOK // TM,)
    out = pl.pallas_call(
        _body,
        grid=grid,
        in_specs=[
            pl.BlockSpec((TM, H * D_HEAD), lambda i: (i, 0)),
            pl.BlockSpec((TM, H), lambda i: (i, 0)),
            pl.BlockSpec((TM, H), lambda i: (i, 0)),
            pl.BlockSpec((E // 2, 2 * D_HEAD, D_MODEL), lambda i: (0, 0, 0)),
            pl.BlockSpec((E, D_MODEL), lambda i: (0, 0)),
        ],
        out_specs=pl.BlockSpec((TM, D_MODEL), lambda i: (i, 0)),
        out_shape=jax.ShapeDtypeStruct((N_TOK, D_MODEL), jnp.float32),
        scratch_shapes=[pltpu.VMEM((E // 2, 2 * D_HEAD, D_MODEL), jnp.bfloat16)],
        compiler_params=pltpu.CompilerParams(
            dimension_semantics=("arbitrary",),
        ),
        interpret=interpret,
    )(x2, sel, p, w4, b)
    return out.reshape(B, S, D_MODEL)
